# Initial kernel scaffold; baseline (speedup 1.0000x reference)
import jax
import jax.numpy as jnp
from jax.experimental import pallas as pl


def kernel(x, edge_index, W1, b1, W2, b2):
    raise NotImplementedError("write your pallas kernel here")


# speedup vs baseline: 1.5467x; 1.5467x over previous
import jax
import jax.numpy as jnp
from jax import lax
from jax.experimental import pallas as pl
from jax.experimental.pallas import tpu as pltpu

N = 10000
D = 128
E = 320000
CH = 512
STEPS = E // CH
BLK = 1000
NBLK = N // BLK


def _tc_degree(dst):
    def body(dst_ref, deg_ref):
        @pl.when(pl.program_id(0) == 0)
        def _():
            deg_ref[...] = jnp.ones_like(deg_ref)

        def loop(i, carry):
            d = dst_ref[i]
            deg_ref[pl.ds(d, 1), :] = deg_ref[pl.ds(d, 1), :] + 1.0
            return carry

        lax.fori_loop(0, CH, loop, 0)

    return pl.pallas_call(
        body,
        grid=(STEPS,),
        in_specs=[pl.BlockSpec((CH,), lambda i: (i,), memory_space=pltpu.SMEM)],
        out_specs=pl.BlockSpec((N, 1), lambda i: (0, 0)),
        out_shape=jax.ShapeDtypeStruct((N, 1), jnp.float32),
    )(dst)


def _tc_aggregate(g, src, dst):
    def body(src_ref, dst_ref, g_ref, p_ref):
        @pl.when(pl.program_id(0) == 0)
        def _():
            p_ref[...] = jnp.zeros_like(p_ref)

        def loop(i, carry):
            s = src_ref[i]
            d = dst_ref[i]
            p_ref[pl.ds(d, 1), :] = p_ref[pl.ds(d, 1), :] + g_ref[pl.ds(s, 1), :]
            return carry

        lax.fori_loop(0, CH, loop, 0)

    return pl.pallas_call(
        body,
        grid=(STEPS,),
        in_specs=[
            pl.BlockSpec((CH,), lambda i: (i,), memory_space=pltpu.SMEM),
            pl.BlockSpec((CH,), lambda i: (i,), memory_space=pltpu.SMEM),
            pl.BlockSpec((N, D), lambda i: (0, 0)),
        ],
        out_specs=pl.BlockSpec((N, D), lambda i: (0, 0)),
        out_shape=jax.ShapeDtypeStruct((N, D), jnp.float32),
    )(src, dst, g)


def _tc_layer1(deg, x, W1):
    def body(deg_ref, x_ref, w_ref, g_ref, dinv_ref):
        dinv = lax.rsqrt(deg_ref[...])
        h = jnp.dot(x_ref[...], w_ref[...], preferred_element_type=jnp.float32)
        g_ref[...] = h * dinv
        dinv_ref[...] = dinv

    return pl.pallas_call(
        body,
        grid=(NBLK,),
        in_specs=[
            pl.BlockSpec((BLK, 1), lambda i: (i, 0)),
            pl.BlockSpec((BLK, D), lambda i: (i, 0)),
            pl.BlockSpec((D, D), lambda i: (0, 0)),
        ],
        out_specs=[
            pl.BlockSpec((BLK, D), lambda i: (i, 0)),
            pl.BlockSpec((BLK, 1), lambda i: (i, 0)),
        ],
        out_shape=[
            jax.ShapeDtypeStruct((N, D), jnp.float32),
            jax.ShapeDtypeStruct((N, 1), jnp.float32),
        ],
    )(deg, x, W1)


def _tc_layer2(p, g1, dinv, W2, b1):
    def body(p_ref, g_ref, dinv_ref, w_ref, b_ref, g2_ref):
        acc = p_ref[...] + g_ref[...]
        r = jnp.maximum(acc * dinv_ref[...] + b_ref[...], 0.0)
        h2 = jnp.dot(r, w_ref[...], preferred_element_type=jnp.float32)
        g2_ref[...] = h2 * dinv_ref[...]

    return pl.pallas_call(
        body,
        grid=(NBLK,),
        in_specs=[
            pl.BlockSpec((BLK, D), lambda i: (i, 0)),
            pl.BlockSpec((BLK, D), lambda i: (i, 0)),
            pl.BlockSpec((BLK, 1), lambda i: (i, 0)),
            pl.BlockSpec((D, D), lambda i: (0, 0)),
            pl.BlockSpec((1, D), lambda i: (0, 0)),
        ],
        out_specs=pl.BlockSpec((BLK, D), lambda i: (i, 0)),
        out_shape=jax.ShapeDtypeStruct((N, D), jnp.float32),
    )(p, g1, dinv, W2, b1)


def _tc_layer3(p, g2, dinv, b2):
    def body(p_ref, g_ref, dinv_ref, b_ref, o_ref):
        acc = p_ref[...] + g_ref[...]
        o = acc * dinv_ref[...] + b_ref[...]
        m = jnp.max(o, axis=1, keepdims=True)
        e = jnp.exp(o - m)
        lse = jnp.log(jnp.sum(e, axis=1, keepdims=True)) + m
        o_ref[...] = o - lse

    return pl.pallas_call(
        body,
        grid=(NBLK,),
        in_specs=[
            pl.BlockSpec((BLK, D), lambda i: (i, 0)),
            pl.BlockSpec((BLK, D), lambda i: (i, 0)),
            pl.BlockSpec((BLK, 1), lambda i: (i, 0)),
            pl.BlockSpec((1, D), lambda i: (0, 0)),
        ],
        out_specs=pl.BlockSpec((BLK, D), lambda i: (i, 0)),
        out_shape=jax.ShapeDtypeStruct((N, D), jnp.float32),
    )(p, g2, dinv, b2)


def kernel(x, edge_index, W1, b1, W2, b2):
    src = edge_index[0].astype(jnp.int32)
    dst = edge_index[1].astype(jnp.int32)

    deg = _tc_degree(dst)
    g1, dinv = _tc_layer1(deg, x, W1)
    p1 = _tc_aggregate(g1, src, dst)
    g2 = _tc_layer2(p1, g1, dinv, W2, b1.reshape(1, D))
    p2 = _tc_aggregate(g2, src, dst)
    return _tc_layer3(p2, g2, dinv, b2.reshape(1, D))
